# Initial kernel scaffold; baseline (speedup 1.0000x reference)
import jax
import jax.numpy as jnp
from jax.experimental import pallas as pl


def kernel(decoder_output, tgt_in_idx, W, b):
    raise NotImplementedError("write your pallas kernel here")


# speedup vs baseline: 6.2086x; 6.2086x over previous
import jax
import jax.numpy as jnp
from jax.experimental import pallas as pl
from jax.experimental.pallas import tpu as pltpu

_B, _S, _D, _C = 128, 32, 64, 100000
_BS = _B * _S
_BC = 512
_NC = (_C + _BC - 1) // _BC
_NEG_INF = float("-inf")


def _block_mask(tgt, ci):
    cols = ci * _BC + jax.lax.broadcasted_iota(jnp.int32, (1, _BC), 1)
    hit = jnp.full((_B, _BC), _S, dtype=jnp.int32)
    for j in range(_S):
        tj = tgt[:, j : j + 1]
        hit = jnp.minimum(hit, jnp.where(tj == cols, j, _S))
    s_iota = jax.lax.broadcasted_iota(jnp.int32, (_B, _S, _BC), 1)
    mask = (s_iota >= hit[:, None, :]).reshape(_BS, _BC)
    colmask = (cols < 2) | (cols >= _C)
    return mask | colmask


def _logits_block(x_ref, w_ref, b_ref):
    x = x_ref[...]
    w = w_ref[...]
    acc = jax.lax.dot_general(
        x, w, (((1,), (1,)), ((), ())), preferred_element_type=jnp.float32
    )
    return acc + b_ref[...]


def _stats_kernel(x_ref, w_ref, b_ref, tgt_ref, denom_ref, m_scr, l_scr):
    ci = pl.program_id(0)

    @pl.when(ci == 0)
    def _init():
        m_scr[...] = jnp.full((_BS, 1), _NEG_INF, dtype=jnp.float32)
        l_scr[...] = jnp.zeros((_BS, 1), dtype=jnp.float32)

    logits = _logits_block(x_ref, w_ref, b_ref)
    mask = _block_mask(tgt_ref[...], ci)
    logits = jnp.where(mask, _NEG_INF, logits)

    m_old = m_scr[...]
    l_old = l_scr[...]
    bmax = jnp.max(logits, axis=1, keepdims=True)
    m_new = jnp.maximum(m_old, bmax)
    bsum = jnp.sum(jnp.exp(logits - m_new), axis=1, keepdims=True)
    l_new = l_old * jnp.exp(m_old - m_new) + bsum
    m_scr[...] = m_new
    l_scr[...] = l_new

    @pl.when(ci == _NC - 1)
    def _fin():
        denom_ref[...] = m_new + jnp.log(l_new)


def _out_kernel(x_ref, w_ref, b_ref, tgt_ref, denom_ref, out_ref):
    ci = pl.program_id(0)
    logits = _logits_block(x_ref, w_ref, b_ref)
    mask = _block_mask(tgt_ref[...], ci)
    out_ref[...] = jnp.where(mask, _NEG_INF, logits - denom_ref[...])


def kernel(decoder_output, tgt_in_idx, W, b):
    x = decoder_output.reshape(_BS, _D)
    tgt = tgt_in_idx.astype(jnp.int32)
    b2 = b.reshape(1, _C)

    denom = pl.pallas_call(
        _stats_kernel,
        grid=(_NC,),
        in_specs=[
            pl.BlockSpec((_BS, _D), lambda ci: (0, 0)),
            pl.BlockSpec((_BC, _D), lambda ci: (ci, 0)),
            pl.BlockSpec((1, _BC), lambda ci: (0, ci)),
            pl.BlockSpec((_B, _S), lambda ci: (0, 0)),
        ],
        out_specs=pl.BlockSpec((_BS, 1), lambda ci: (0, 0)),
        out_shape=jax.ShapeDtypeStruct((_BS, 1), jnp.float32),
        scratch_shapes=[
            pltpu.VMEM((_BS, 1), jnp.float32),
            pltpu.VMEM((_BS, 1), jnp.float32),
        ],
    )(x, W, b2, tgt)

    out = pl.pallas_call(
        _out_kernel,
        grid=(_NC,),
        in_specs=[
            pl.BlockSpec((_BS, _D), lambda ci: (0, 0)),
            pl.BlockSpec((_BC, _D), lambda ci: (ci, 0)),
            pl.BlockSpec((1, _BC), lambda ci: (0, ci)),
            pl.BlockSpec((_B, _S), lambda ci: (0, 0)),
            pl.BlockSpec((_BS, 1), lambda ci: (0, 0)),
        ],
        out_specs=pl.BlockSpec((_BS, _BC), lambda ci: (0, ci)),
        out_shape=jax.ShapeDtypeStruct((_BS, _C), jnp.float32),
    )(x, W, b2, tgt, denom)

    return out.reshape(_B, _S, _C)


# speedup vs baseline: 6.3053x; 1.0156x over previous
import jax
import jax.numpy as jnp
from jax.experimental import pallas as pl
from jax.experimental.pallas import tpu as pltpu

_B, _S, _D, _C = 128, 32, 64, 100000
_BS = _B * _S
_BC = 512
_NC = (_C + _BC - 1) // _BC
_NEG_INF = float("-inf")


def _block_mask(tgt, ci):
    cols = ci * _BC + jax.lax.broadcasted_iota(jnp.int32, (1, _BC), 1)
    hit = jnp.full((_B, _BC), _S, dtype=jnp.int32)
    for j in range(_S):
        tj = tgt[:, j : j + 1]
        hit = jnp.minimum(hit, jnp.where(tj == cols, j, _S))
    s_iota = jax.lax.broadcasted_iota(jnp.int32, (_B, _S, _BC), 1)
    mask = (s_iota >= hit[:, None, :]).reshape(_BS, _BC)
    colmask = (cols < 2) | (cols >= _C)
    return mask | colmask


def _logits_block(x_ref, w_ref, b_ref):
    x = x_ref[...]
    w = w_ref[...]
    acc = jax.lax.dot_general(
        x, w, (((1,), (1,)), ((), ())), preferred_element_type=jnp.float32
    )
    return acc + b_ref[...]


def _stats_kernel(x_ref, w_ref, b_ref, tgt_ref, denom_ref, m_scr, l_scr):
    ci = pl.program_id(0)

    @pl.when(ci == 0)
    def _init():
        m_scr[...] = jnp.full((_BS, 1), _NEG_INF, dtype=jnp.float32)
        l_scr[...] = jnp.zeros((_BS, 1), dtype=jnp.float32)

    logits = _logits_block(x_ref, w_ref, b_ref)
    mask = _block_mask(tgt_ref[...], ci)
    logits = jnp.where(mask, _NEG_INF, logits)

    m_old = m_scr[...]
    l_old = l_scr[...]
    bmax = jnp.max(logits, axis=1, keepdims=True)
    m_new = jnp.maximum(m_old, bmax)
    bsum = jnp.sum(jnp.exp(logits - m_new), axis=1, keepdims=True)
    l_new = l_old * jnp.exp(m_old - m_new) + bsum
    m_scr[...] = m_new
    l_scr[...] = l_new

    @pl.when(ci == _NC - 1)
    def _fin():
        denom_ref[...] = m_new + jnp.log(l_new)


def _out_kernel(x_ref, w_ref, b_ref, tgt_ref, denom_ref, out_ref):
    ci = pl.program_id(0)
    logits = _logits_block(x_ref, w_ref, b_ref)
    mask = _block_mask(tgt_ref[...], ci)
    out_ref[...] = jnp.where(mask, _NEG_INF, logits - denom_ref[...])


def kernel(decoder_output, tgt_in_idx, W, b):
    x = decoder_output.reshape(_BS, _D).astype(jnp.bfloat16)
    W = W.astype(jnp.bfloat16)
    tgt = tgt_in_idx.astype(jnp.int32)
    b2 = b.reshape(1, _C)

    denom = pl.pallas_call(
        _stats_kernel,
        grid=(_NC,),
        in_specs=[
            pl.BlockSpec((_BS, _D), lambda ci: (0, 0)),
            pl.BlockSpec((_BC, _D), lambda ci: (ci, 0)),
            pl.BlockSpec((1, _BC), lambda ci: (0, ci)),
            pl.BlockSpec((_B, _S), lambda ci: (0, 0)),
        ],
        out_specs=pl.BlockSpec((_BS, 1), lambda ci: (0, 0)),
        out_shape=jax.ShapeDtypeStruct((_BS, 1), jnp.float32),
        scratch_shapes=[
            pltpu.VMEM((_BS, 1), jnp.float32),
            pltpu.VMEM((_BS, 1), jnp.float32),
        ],
    )(x, W, b2, tgt)

    out = pl.pallas_call(
        _out_kernel,
        grid=(_NC,),
        in_specs=[
            pl.BlockSpec((_BS, _D), lambda ci: (0, 0)),
            pl.BlockSpec((_BC, _D), lambda ci: (ci, 0)),
            pl.BlockSpec((1, _BC), lambda ci: (0, ci)),
            pl.BlockSpec((_B, _S), lambda ci: (0, 0)),
            pl.BlockSpec((_BS, 1), lambda ci: (0, 0)),
        ],
        out_specs=pl.BlockSpec((_BS, _BC), lambda ci: (0, ci)),
        out_shape=jax.ShapeDtypeStruct((_BS, _C), jnp.float32),
    )(x, W, b2, tgt, denom)

    return out.reshape(_B, _S, _C)
